# baseline (speedup 1.0000x reference)
import jax
import jax.numpy as jnp
from jax import lax
from jax.experimental import pallas as pl
from jax.experimental.pallas import tpu as pltpu
from jax.experimental.pallas import tpu_sc as plsc

_NC, _NS, _L = 2, 16, 16


def _sc_fill(*, b, d, h, w):
    nw = _NC * _NS
    ipw = h // nw
    d_half = d // 2
    nv = d_half // _L
    mesh = plsc.VectorSubcoreMesh(core_axis_name="c", subcore_axis_name="s")

    def body(col_hbm, row_hbm, o_hbm, srow, buf0, buf1, sem0, sem1, sem2):
        wid = lax.axis_index("s") * _NC + lax.axis_index("c")
        i0 = wid * ipw
        bufs, sems = (buf0, buf1), (sem0, sem1)

        start = pl.multiple_of(jnp.minimum(i0 - i0 % 8, h - 16), 8)
        off = i0 - start
        cp = pltpu.make_async_copy(row_hbm.at[pl.ds(start, 16)], srow, sem2)
        cp.start()
        col_cp = [
            pltpu.make_async_copy(col_hbm, bf.at[:, pl.ds(0, d_half)], sm)
            for bf, sm in ((buf0, sem0), (buf1, sem1))
        ]
        col_cp[0].start()
        col_cp[1].start()
        cp.wait()

        waits = {}
        for k in range(ipw):
            buf, sem = bufs[k % 2], sems[k % 2]
            if k >= 2:
                for cp in waits[k - 2]:
                    cp.wait()
            vs = [srow.at[off + k][pl.ds(jj * _L, _L)] for jj in range(nv)]

            def rowfill(j, carry):
                for jj in range(nv):
                    buf.at[j][pl.ds(d_half + jj * _L, _L)] = vs[jj]
                return carry

            lax.fori_loop(0, h, rowfill, 0, unroll=False)
            if k < 2:
                col_cp[k].wait()
            waits[k] = []
            for bb in range(b):
                cp = pltpu.make_async_copy(buf, o_hbm.at[bb, i0 + k], sem)
                cp.start()
                waits[k].append(cp)
        for k in (ipw - 2, ipw - 1):
            for cp in waits[k]:
                cp.wait()

    return pl.kernel(
        body,
        out_type=jax.ShapeDtypeStruct((b, h, w, d), jnp.float32),
        mesh=mesh,
        scratch_types=[
            pltpu.VMEM((16, d_half), jnp.float32),
            pltpu.VMEM((w, d), jnp.float32),
            pltpu.VMEM((w, d), jnp.float32),
            pltpu.SemaphoreType.DMA,
            pltpu.SemaphoreType.DMA,
            pltpu.SemaphoreType.DMA,
        ],
        compiler_params=pltpu.CompilerParams(use_tc_tiling_on_sc=True),
    )


def kernel(x, row_embed, col_embed):
    b = x.shape[0]
    h, w = x.shape[2], x.shape[3]
    d_half = row_embed.shape[1]
    d = 2 * d_half
    out_bhwd = _sc_fill(b=b, d=d, h=h, w=w)(col_embed[:w], row_embed[:h])
    return jnp.transpose(out_bhwd, (0, 3, 1, 2)).astype(x.dtype)


# speedup vs baseline: 1.0596x; 1.0596x over previous
import jax
import jax.numpy as jnp
from jax import lax
from jax.experimental import pallas as pl
from jax.experimental.pallas import tpu as pltpu
from jax.experimental.pallas import tpu_sc as plsc

_NC, _NS, _L = 2, 16, 16


def _sc_fill(*, b, d, h, w):
    nw = _NC * _NS
    ipw = h // nw
    d_half = d // 2
    nv = d_half // _L
    mesh = plsc.VectorSubcoreMesh(core_axis_name="c", subcore_axis_name="s")

    def body(col_hbm, row_hbm, o_hbm, srow, buf0, buf1, sem0, sem1, sem2):
        wid = lax.axis_index("s") * _NC + lax.axis_index("c")
        i0 = wid * ipw
        bufs, sems = (buf0, buf1), (sem0, sem1)

        start = pl.multiple_of(jnp.minimum(i0 - i0 % 8, h - 16), 8)
        off = i0 - start
        cp = pltpu.make_async_copy(row_hbm.at[pl.ds(start, 16)], srow, sem2)
        cp.start()
        col_cp = [
            pltpu.make_async_copy(col_hbm, bf.at[:, pl.ds(0, d_half)], sm)
            for bf, sm in ((buf0, sem0), (buf1, sem1))
        ]
        col_cp[0].start()
        col_cp[1].start()
        cp.wait()
        col_cp[0].wait()
        col_cp[1].wait()

        waits = {}
        for k in range(ipw):
            buf, sem = bufs[k % 2], sems[k % 2]
            if k >= 2:
                for cp in waits[k - 2]:
                    cp.wait()
            vs = [srow.at[off + k][pl.ds(jj * _L, _L)] for jj in range(nv)]

            def rowfill(j, carry):
                for jj in range(nv):
                    buf.at[j][pl.ds(d_half + jj * _L, _L)] = vs[jj]
                return carry

            lax.fori_loop(0, h, rowfill, 0, unroll=False)
            waits[k] = []
            for bb in range(b):
                cp = pltpu.make_async_copy(buf, o_hbm.at[bb, i0 + k], sem)
                cp.start()
                waits[k].append(cp)
        for k in (ipw - 2, ipw - 1):
            for cp in waits[k]:
                cp.wait()

    return pl.kernel(
        body,
        out_type=jax.ShapeDtypeStruct((b, h, w, d), jnp.float32),
        mesh=mesh,
        scratch_types=[
            pltpu.VMEM((16, d_half), jnp.float32),
            pltpu.VMEM((w, d), jnp.float32),
            pltpu.VMEM((w, d), jnp.float32),
            pltpu.SemaphoreType.DMA,
            pltpu.SemaphoreType.DMA,
            pltpu.SemaphoreType.DMA,
        ],
        compiler_params=pltpu.CompilerParams(use_tc_tiling_on_sc=True),
    )


def kernel(x, row_embed, col_embed):
    b = x.shape[0]
    h, w = x.shape[2], x.shape[3]
    d_half = row_embed.shape[1]
    d = 2 * d_half
    out_bhwd = _sc_fill(b=b, d=d, h=h, w=w)(col_embed[:w], row_embed[:h])
    return jnp.transpose(out_bhwd, (0, 3, 1, 2)).astype(x.dtype)
